# baseline (speedup 1.0000x reference)
import functools

import jax
import jax.numpy as jnp
from jax import lax
from jax.experimental import pallas as pl
from jax.experimental.pallas import tpu as pltpu
from jax.experimental.pallas import tpu_sc as plsc

NC = 2
NS = 16
NW = NC * NS
IPG = 128


@jax.jit
def _gather(table, idxT3):
    d = table.shape[1]
    F = idxT3.shape[0]
    B = idxT3.shape[1] * idxT3.shape[2]
    bpw = B // NW
    ng = bpw // IPG

    mesh = plsc.VectorSubcoreMesh(core_axis_name="c", subcore_axis_name="s")

    @functools.partial(
        pl.kernel,
        out_type=jax.ShapeDtypeStruct((F, d, B), jnp.float32),
        mesh=mesh,
        scratch_types=[
            pltpu.VMEM((ng, IPG), jnp.int32),
            pltpu.VMEM((bpw, d), jnp.float32),
            pltpu.VMEM((d, bpw), jnp.float32),
            pltpu.SemaphoreType.DMA,
        ],
        compiler_params=pltpu.CompilerParams(
            use_tc_tiling_on_sc=False, needs_layout_passes=False),
    )
    def k(table_hbm, idx_hbm, out_hbm, idx_v, rows_v, rowsT_v, sem):
        wid = lax.axis_index("s") * NC + lax.axis_index("c")
        b0 = wid * bpw
        iota16 = lax.iota(jnp.int32, 16)

        @pl.loop(0, F)
        def per_f(f):
            pltpu.sync_copy(idx_hbm.at[f, pl.ds(wid * ng, ng)], idx_v)
            for j in range(ng):
                pltpu.async_copy(
                    table_hbm.at[idx_v.at[j]],
                    rows_v.at[pl.ds(j * IPG, IPG)], sem)
            for j in range(ng):
                pltpu.make_async_copy(
                    table_hbm.at[idx_v.at[j]],
                    rows_v.at[pl.ds(j * IPG, IPG)], sem).wait()

            @pl.loop(0, bpw // 16)
            def per_c(c):
                rows16 = c * 16 + iota16
                for e in range(d):
                    vals = plsc.load_gather(
                        rows_v, [rows16, jnp.full((16,), e, jnp.int32)])
                    rowsT_v[e, pl.ds(c * 16, 16)] = vals

            pltpu.sync_copy(rowsT_v, out_hbm.at[f, :, pl.ds(b0, bpw)])

    return k(table, idxT3)


def kernel(table, idx):
    B, F = idx.shape
    d = table.shape[1]
    idxT3 = jnp.transpose(idx).reshape(F, B // IPG, IPG)
    out_p = _gather(table, idxT3)
    return jnp.transpose(out_p, (2, 0, 1))


# speedup vs baseline: 1.2490x; 1.2490x over previous
import functools

import jax
import jax.numpy as jnp
from jax import lax
from jax.experimental import pallas as pl
from jax.experimental.pallas import tpu as pltpu
from jax.experimental.pallas import tpu_sc as plsc

NC = 2
NS = 16
NW = NC * NS
IPG = 128


@jax.jit
def _gather(table, idxT3):
    d = table.shape[1]
    F = idxT3.shape[0]
    B = idxT3.shape[1] * idxT3.shape[2]
    bpw = B // NW
    ng = bpw // IPG

    mesh = plsc.VectorSubcoreMesh(core_axis_name="c", subcore_axis_name="s")

    @functools.partial(
        pl.kernel,
        out_type=jax.ShapeDtypeStruct((F, B, d), jnp.float32),
        mesh=mesh,
        scratch_types=[
            pltpu.VMEM((2, ng, IPG), jnp.int32),
            pltpu.VMEM((2, bpw, d), jnp.float32),
            pltpu.SemaphoreType.DMA((2,)),
            pltpu.SemaphoreType.DMA((2,)),
        ],
        compiler_params=pltpu.CompilerParams(
            use_tc_tiling_on_sc=False, needs_layout_passes=False),
    )
    def k(table_hbm, idx_hbm, out_hbm, idx_v, rows_v, gsem, wsem):
        wid = lax.axis_index("s") * NC + lax.axis_index("c")
        b0 = wid * bpw

        def load_idx(f, p):
            pltpu.sync_copy(idx_hbm.at[f, pl.ds(wid * ng, ng)], idx_v.at[p])

        def gathers(p):
            for j in range(ng):
                pltpu.async_copy(
                    table_hbm.at[idx_v.at[p].at[j]],
                    rows_v.at[p].at[pl.ds(j * IPG, IPG)], gsem.at[p])

        def wait_gathers(p):
            for j in range(ng):
                pltpu.make_async_copy(
                    table_hbm.at[idx_v.at[p].at[j]],
                    rows_v.at[p].at[pl.ds(j * IPG, IPG)], gsem.at[p]).wait()

        def writeback(f, p):
            return pltpu.make_async_copy(
                rows_v.at[p], out_hbm.at[f, pl.ds(b0, bpw)], wsem.at[p])

        load_idx(0, 0)
        gathers(0)

        @pl.loop(0, F, step=2)
        def per_f2(f2):
            for p in range(2):
                f = f2 + p
                wait_gathers(p)
                writeback(f, p).start()

                @pl.when(f + 1 < F)
                def _():
                    load_idx(f + 1, 1 - p)

                    @pl.when(f >= 1)
                    def _():
                        writeback(f - 1, 1 - p).wait()

                    gathers(1 - p)

        writeback(F - 2, 0).wait()
        writeback(F - 1, 1).wait()

    return k(table, idxT3)


def kernel(table, idx):
    B, F = idx.shape
    d = table.shape[1]
    idxT3 = jnp.transpose(idx).reshape(F, B // IPG, IPG)
    out_p = _gather(table, idxT3)
    return jnp.transpose(out_p, (1, 0, 2))


# speedup vs baseline: 1.2775x; 1.0228x over previous
import functools

import jax
import jax.numpy as jnp
from jax import lax
from jax.experimental import pallas as pl
from jax.experimental.pallas import tpu as pltpu
from jax.experimental.pallas import tpu_sc as plsc

NC = 2
NS = 16
NW = NC * NS
IPG = 128
NSLOT = 3


@jax.jit
def _gather(table, idxT3):
    d = table.shape[1]
    F = idxT3.shape[0]
    B = idxT3.shape[1] * idxT3.shape[2]
    bpw = B // NW
    ng = bpw // IPG

    mesh = plsc.VectorSubcoreMesh(core_axis_name="c", subcore_axis_name="s")

    @functools.partial(
        pl.kernel,
        out_type=jax.ShapeDtypeStruct((F, B, d), jnp.float32),
        mesh=mesh,
        scratch_types=[
            pltpu.VMEM((F * ng, IPG), jnp.int32),
            pltpu.VMEM((NSLOT, bpw, d), jnp.float32),
            pltpu.SemaphoreType.DMA((NSLOT,)),
            pltpu.SemaphoreType.DMA((NSLOT,)),
        ],
        compiler_params=pltpu.CompilerParams(
            use_tc_tiling_on_sc=False, needs_layout_passes=False),
    )
    def k(table_hbm, idx_hbm, out_hbm, idx_v, rows_v, gsem, wsem):
        wid = lax.axis_index("s") * NC + lax.axis_index("c")
        b0 = wid * bpw

        for f in range(F):
            pltpu.async_copy(
                idx_hbm.at[f, pl.ds(wid * ng, ng)],
                idx_v.at[pl.ds(f * ng, ng)], gsem.at[0])
        for f in range(F):
            pltpu.make_async_copy(
                idx_hbm.at[f, pl.ds(wid * ng, ng)],
                idx_v.at[pl.ds(f * ng, ng)], gsem.at[0]).wait()

        def gathers(f, p):
            for j in range(ng):
                pltpu.async_copy(
                    table_hbm.at[idx_v.at[f * ng + j]],
                    rows_v.at[p].at[pl.ds(j * IPG, IPG)], gsem.at[p])

        def wait_gathers(f, p):
            for j in range(ng):
                pltpu.make_async_copy(
                    table_hbm.at[idx_v.at[f * ng + j]],
                    rows_v.at[p].at[pl.ds(j * IPG, IPG)], gsem.at[p]).wait()

        def writeback(f, p):
            return pltpu.make_async_copy(
                rows_v.at[p], out_hbm.at[f, pl.ds(b0, bpw)], wsem.at[p])

        gathers(0, 0)
        gathers(1, 1)
        for f in range(F):
            p = f % NSLOT
            wait_gathers(f, p)
            writeback(f, p).start()
            if f + 2 < F:
                if f >= 1:
                    writeback(f - 1, (f + 2) % NSLOT).wait()
                gathers(f + 2, (f + 2) % NSLOT)
        writeback(F - 3, (F - 3) % NSLOT).wait()
        writeback(F - 2, (F - 2) % NSLOT).wait()
        writeback(F - 1, (F - 1) % NSLOT).wait()

    return k(table, idxT3)


def kernel(table, idx):
    B, F = idx.shape
    d = table.shape[1]
    idxT3 = jnp.transpose(idx).reshape(F, B // IPG, IPG)
    out_p = _gather(table, idxT3)
    return jnp.transpose(out_p, (1, 0, 2))


# speedup vs baseline: 1.2789x; 1.0011x over previous
import functools

import jax
import jax.numpy as jnp
from jax import lax
from jax.experimental import pallas as pl
from jax.experimental.pallas import tpu as pltpu
from jax.experimental.pallas import tpu_sc as plsc

NC = 2
NS = 16
NW = NC * NS
IPG = 128
NSLOT = 3


@jax.jit
def _gather(table, idxT3):
    d = table.shape[1]
    F = idxT3.shape[0]
    B = idxT3.shape[1] * idxT3.shape[2]
    bpw = B // NW
    ng = bpw // IPG

    mesh = plsc.VectorSubcoreMesh(core_axis_name="c", subcore_axis_name="s")

    @functools.partial(
        pl.kernel,
        out_type=jax.ShapeDtypeStruct((F, B, d), jnp.float32),
        mesh=mesh,
        scratch_types=[
            pltpu.VMEM((F * ng, IPG), jnp.int32),
            pltpu.VMEM((NSLOT, bpw, d), jnp.float32),
            pltpu.SemaphoreType.DMA((NSLOT,)),
            pltpu.SemaphoreType.DMA((NSLOT,)),
        ],
        compiler_params=pltpu.CompilerParams(
            use_tc_tiling_on_sc=False, needs_layout_passes=False),
    )
    def k(table_hbm, idx_hbm, out_hbm, idx_v, rows_v, gsem, wsem):
        wid = lax.axis_index("s") * NC + lax.axis_index("c")
        b0 = wid * bpw

        for f in range(F):
            pltpu.async_copy(
                idx_hbm.at[f, pl.ds(wid * ng, ng)],
                idx_v.at[pl.ds(f * ng, ng)], gsem.at[0])
        for f in range(F):
            pltpu.make_async_copy(
                idx_hbm.at[f, pl.ds(wid * ng, ng)],
                idx_v.at[pl.ds(f * ng, ng)], gsem.at[0]).wait()

        def gathers(f, p):
            for j in range(ng):
                pltpu.async_copy(
                    table_hbm.at[idx_v.at[f * ng + j]],
                    rows_v.at[p].at[pl.ds(j * IPG, IPG)], gsem.at[p])

        def wait_gathers(f, p):
            for j in range(ng):
                pltpu.make_async_copy(
                    table_hbm.at[idx_v.at[f * ng + j]],
                    rows_v.at[p].at[pl.ds(j * IPG, IPG)], gsem.at[p]).wait()

        def writeback(f, p):
            return pltpu.make_async_copy(
                rows_v.at[p], out_hbm.at[f, pl.ds(b0, bpw)], wsem.at[p])

        gathers(0, 0)
        gathers(1, 1)
        for f in range(F):
            p = f % NSLOT
            wait_gathers(f, p)
            writeback(f, p).start()
            if f + 2 < F:
                if f >= 1:
                    writeback(f - 1, (f + 2) % NSLOT).wait()
                gathers(f + 2, (f + 2) % NSLOT)
        writeback(F - 3, (F - 3) % NSLOT).wait()
        writeback(F - 2, (F - 2) % NSLOT).wait()
        writeback(F - 1, (F - 1) % NSLOT).wait()

    return k(table, idxT3)


def kernel(table, idx):
    B, F = idx.shape
    V, d = table.shape
    idxT3 = jnp.transpose(idx).reshape(F, B // IPG, IPG)
    table128 = lax.optimization_barrier(jnp.reshape(table, (V * d // 128, 128)))
    table_lin = jnp.reshape(table128, (V, d))
    out_p = _gather(table_lin, idxT3)
    return jnp.transpose(out_p, (1, 0, 2))


# speedup vs baseline: 1.4243x; 1.1137x over previous
import functools

import jax
import jax.numpy as jnp
from jax import lax
from jax.experimental import pallas as pl
from jax.experimental.pallas import tpu as pltpu
from jax.experimental.pallas import tpu_sc as plsc

NC = 2
NS = 16
NW = NC * NS
IPG = 128
NSLOT = 3


def _repack(tableT):
    d, V = tableT.shape
    C = 2048
    G = -(-V // C)
    C4 = C * d // 128
    npk = 128 // d

    def body(x_ref, o_ref):
        x = x_ref[...]
        o_ref[...] = jnp.concatenate(
            [jnp.transpose(x[:, a * C4:(a + 1) * C4]) for a in range(npk)],
            axis=1)

    return pl.pallas_call(
        body,
        grid=(G,),
        in_specs=[pl.BlockSpec((d, C), lambda g: (0, g))],
        out_specs=pl.BlockSpec((C4, 128), lambda g: (g, 0)),
        out_shape=jax.ShapeDtypeStruct((G * C4, 128), jnp.float32),
    )(tableT)


@jax.jit
def _gather(table, idxT3):
    d = table.shape[1]
    F = idxT3.shape[0]
    B = idxT3.shape[1] * idxT3.shape[2]
    bpw = B // NW
    ng = bpw // IPG

    mesh = plsc.VectorSubcoreMesh(core_axis_name="c", subcore_axis_name="s")

    @functools.partial(
        pl.kernel,
        out_type=jax.ShapeDtypeStruct((F, B, d), jnp.float32),
        mesh=mesh,
        scratch_types=[
            pltpu.VMEM((F * ng, IPG), jnp.int32),
            pltpu.VMEM((NSLOT, bpw, d), jnp.float32),
            pltpu.SemaphoreType.DMA((NSLOT,)),
            pltpu.SemaphoreType.DMA((NSLOT,)),
        ],
        compiler_params=pltpu.CompilerParams(
            use_tc_tiling_on_sc=False, needs_layout_passes=False),
    )
    def k(table_hbm, idx_hbm, out_hbm, idx_v, rows_v, gsem, wsem):
        wid = lax.axis_index("s") * NC + lax.axis_index("c")
        b0 = wid * bpw

        for f in range(F):
            pltpu.async_copy(
                idx_hbm.at[f, pl.ds(wid * ng, ng)],
                idx_v.at[pl.ds(f * ng, ng)], gsem.at[0])
        for f in range(F):
            pltpu.make_async_copy(
                idx_hbm.at[f, pl.ds(wid * ng, ng)],
                idx_v.at[pl.ds(f * ng, ng)], gsem.at[0]).wait()

        def gathers(f, p):
            for j in range(ng):
                pltpu.async_copy(
                    table_hbm.at[idx_v.at[f * ng + j]],
                    rows_v.at[p].at[pl.ds(j * IPG, IPG)], gsem.at[p])

        def wait_gathers(f, p):
            for j in range(ng):
                pltpu.make_async_copy(
                    table_hbm.at[idx_v.at[f * ng + j]],
                    rows_v.at[p].at[pl.ds(j * IPG, IPG)], gsem.at[p]).wait()

        def writeback(f, p):
            return pltpu.make_async_copy(
                rows_v.at[p], out_hbm.at[f, pl.ds(b0, bpw)], wsem.at[p])

        gathers(0, 0)
        gathers(1, 1)
        for f in range(F):
            p = f % NSLOT
            wait_gathers(f, p)
            writeback(f, p).start()
            if f + 2 < F:
                if f >= 1:
                    writeback(f - 1, (f + 2) % NSLOT).wait()
                gathers(f + 2, (f + 2) % NSLOT)
        writeback(F - 3, (F - 3) % NSLOT).wait()
        writeback(F - 2, (F - 2) % NSLOT).wait()
        writeback(F - 1, (F - 1) % NSLOT).wait()

    return k(table, idxT3)


def kernel(table, idx):
    B, F = idx.shape
    V, d = table.shape
    l = idx % 2048
    idxR = idx - l + 4 * (l % 512) + l // 512
    idxT3 = jnp.transpose(idxR).reshape(F, B // IPG, IPG)
    packed = _repack(jnp.transpose(table))
    table_lin = jnp.reshape(packed, (packed.shape[0] * (128 // d), d))
    out_p = _gather(table_lin, idxT3)
    return jnp.transpose(out_p, (1, 0, 2))


# speedup vs baseline: 1.9005x; 1.3343x over previous
import functools

import jax
import jax.numpy as jnp
from jax import lax
from jax.experimental import pallas as pl
from jax.experimental.pallas import tpu as pltpu
from jax.experimental.pallas import tpu_sc as plsc

NC = 2
NS = 16
NW = NC * NS
IPG = 128
NSLOT = 3
REPACK_C = 32768


def _repack(tableT):
    d, V = tableT.shape
    C = REPACK_C
    G = -(-V // C)
    C4 = C * d // 128
    npk = 128 // d

    def body(x_ref, o_ref):
        x = x_ref[...]
        o_ref[...] = jnp.concatenate(
            [jnp.transpose(x[:, a * C4:(a + 1) * C4]) for a in range(npk)],
            axis=1)

    return pl.pallas_call(
        body,
        grid=(G,),
        in_specs=[pl.BlockSpec((d, C), lambda g: (0, g))],
        out_specs=pl.BlockSpec((C4, 128), lambda g: (g, 0)),
        out_shape=jax.ShapeDtypeStruct((G * C4, 128), jnp.float32),
    )(tableT)


@jax.jit
def _gather(table, idxT3):
    d = table.shape[1]
    F = idxT3.shape[0]
    B = idxT3.shape[1] * idxT3.shape[2]
    bpw = B // NW
    ng = bpw // IPG

    mesh = plsc.VectorSubcoreMesh(core_axis_name="c", subcore_axis_name="s")

    @functools.partial(
        pl.kernel,
        out_type=jax.ShapeDtypeStruct((F, B, d), jnp.float32),
        mesh=mesh,
        scratch_types=[
            pltpu.VMEM((F * ng, IPG), jnp.int32),
            pltpu.VMEM((NSLOT, bpw, d), jnp.float32),
            pltpu.SemaphoreType.DMA((NSLOT,)),
            pltpu.SemaphoreType.DMA((NSLOT,)),
        ],
        compiler_params=pltpu.CompilerParams(
            use_tc_tiling_on_sc=False, needs_layout_passes=False),
    )
    def k(table_hbm, idx_hbm, out_hbm, idx_v, rows_v, gsem, wsem):
        wid = lax.axis_index("s") * NC + lax.axis_index("c")
        b0 = wid * bpw

        for f in range(F):
            pltpu.async_copy(
                idx_hbm.at[f, pl.ds(wid * ng, ng)],
                idx_v.at[pl.ds(f * ng, ng)], gsem.at[0])
        for f in range(F):
            pltpu.make_async_copy(
                idx_hbm.at[f, pl.ds(wid * ng, ng)],
                idx_v.at[pl.ds(f * ng, ng)], gsem.at[0]).wait()

        def gathers(f, p):
            for j in range(ng):
                pltpu.async_copy(
                    table_hbm.at[idx_v.at[f * ng + j]],
                    rows_v.at[p].at[pl.ds(j * IPG, IPG)], gsem.at[p])

        def wait_gathers(f, p):
            for j in range(ng):
                pltpu.make_async_copy(
                    table_hbm.at[idx_v.at[f * ng + j]],
                    rows_v.at[p].at[pl.ds(j * IPG, IPG)], gsem.at[p]).wait()

        def writeback(f, p):
            return pltpu.make_async_copy(
                rows_v.at[p], out_hbm.at[f, pl.ds(b0, bpw)], wsem.at[p])

        gathers(0, 0)
        gathers(1, 1)
        for f in range(F):
            p = f % NSLOT
            wait_gathers(f, p)
            writeback(f, p).start()
            if f + 2 < F:
                if f >= 1:
                    writeback(f - 1, (f + 2) % NSLOT).wait()
                gathers(f + 2, (f + 2) % NSLOT)
        writeback(F - 3, (F - 3) % NSLOT).wait()
        writeback(F - 2, (F - 2) % NSLOT).wait()
        writeback(F - 1, (F - 1) % NSLOT).wait()

    return k(table, idxT3)


def kernel(table, idx):
    B, F = idx.shape
    V, d = table.shape
    npk = 128 // d
    c4 = REPACK_C // npk
    l = idx % REPACK_C
    idxR = idx - l + npk * (l % c4) + l // c4
    idxT3 = jnp.transpose(idxR).reshape(F, B // IPG, IPG)
    packed = _repack(jnp.transpose(table))
    table_lin = jnp.reshape(packed, (packed.shape[0] * (128 // d), d))
    out_p = _gather(table_lin, idxT3)
    return jnp.transpose(out_p, (1, 0, 2))
